# baseline (speedup 1.0000x reference)
import functools

import jax
import jax.numpy as jnp
from jax import lax
from jax.experimental import pallas as pl
from jax.experimental.pallas import tpu as pltpu
from jax.experimental.pallas import tpu_sc as plsc

N = 10000
E = 320000
IN = 128
H = 64

NC = 2
NS = 16
NW = NC * NS
CHUNK = 128
CPW = 80
E_PAD = NW * CPW * CHUNK
EROWS = E // CHUNK
EROWS_PAD = E_PAD // CHUNK
ROWS_PT = 632
N_PAD = NS * ROWS_PT
DUMMY = N_PAD - 1
W2AGG = 16
DW = 16
NBURST = 2
QCH = 16


def _tc1_body(x_ref, ei_ref, wl_ref, wr_ref, b1_ref, pk1_ref,
              srcp_ref, dstp_ref):
    x = x_ref[...]
    p1 = jnp.dot(x, wl_ref[...], preferred_element_type=jnp.float32)
    r1b = jnp.dot(x, wr_ref[...], preferred_element_type=jnp.float32) + b1_ref[...]
    pk1 = jnp.concatenate([p1, r1b], axis=1)
    pk1_ref[...] = jnp.concatenate(
        [pk1, jnp.zeros((N_PAD - N, 2 * H), jnp.float32)], axis=0)
    ei = ei_ref[...].reshape(2, EROWS, CHUNK)
    srcp_ref[...] = jnp.concatenate(
        [ei[0], jnp.zeros((EROWS_PAD - EROWS, CHUNK), jnp.int32)], axis=0)
    dstp_ref[...] = jnp.concatenate(
        [ei[1], jnp.full((EROWS_PAD - EROWS, CHUNK), DUMMY, jnp.int32)], axis=0)


def _tc1(x, edge_index, W1_l, W1_r, b1):
    return pl.pallas_call(
        _tc1_body,
        out_shape=[
            jax.ShapeDtypeStruct((N_PAD, 2 * H), jnp.float32),
            jax.ShapeDtypeStruct((EROWS_PAD, CHUNK), jnp.int32),
            jax.ShapeDtypeStruct((EROWS_PAD, CHUNK), jnp.int32),
        ],
    )(x, edge_index, W1_l, W1_r, b1)


def _tc2_body(so1_ref, pk1_ref, w2l_ref, w2r_ref, b2_ref, p2_ref, aux_ref):
    s1 = so1_ref[0, :N, :H] + so1_ref[1, :N, :H]
    deg = jnp.maximum(so1_ref[0, :N, H:H + 1] + so1_ref[1, :N, H:H + 1], 1.0)
    rdeg = 1.0 / deg
    h = jnp.maximum(s1 * rdeg + pk1_ref[:N, H:], 0.0)
    p2 = jnp.dot(h, w2l_ref[...], preferred_element_type=jnp.float32)
    r2b = jnp.dot(h, w2r_ref[...], preferred_element_type=jnp.float32) + b2_ref[...]
    block = jnp.concatenate(
        [p2, jnp.zeros((N, 2 * H - 1), jnp.float32)], axis=1)
    p2_ref[...] = jnp.concatenate(
        [block, jnp.zeros((N_PAD - N, 2 * H), jnp.float32)], axis=0)
    aux_ref[...] = jnp.concatenate(
        [rdeg, r2b, jnp.zeros((N, 6), jnp.float32)], axis=1)


def _tc2(so1, pk1, W2_l, W2_r, b2):
    return pl.pallas_call(
        _tc2_body,
        out_shape=[
            jax.ShapeDtypeStruct((N_PAD, 2 * H), jnp.float32),
            jax.ShapeDtypeStruct((N, 8), jnp.float32),
        ],
    )(so1, pk1, W2_l, W2_r, b2)


def _tc3_body(s2p_ref, aux_ref, out_ref):
    s2 = s2p_ref[0, :N, 0:1] + s2p_ref[1, :N, 0:1]
    out_ref[...] = s2 * aux_ref[:, 0:1] + aux_ref[:, 1:2]


def _tc3(s2p, aux):
    return pl.pallas_call(
        _tc3_body,
        out_shape=jax.ShapeDtypeStruct((N, 1), jnp.float32),
    )(s2p, aux)


_MESH = plsc.VectorSubcoreMesh(
    core_axis_name="c", subcore_axis_name="s", num_cores=NC, num_subcores=NS)


def _fill(buf, w, val):
    def _frow(i, _):
        def _fcol(j, _):
            buf[i, pl.ds(j * 16, 16)] = jnp.full((16,), val, jnp.float32)
            return 0
        return lax.fori_loop(0, w // 16, _fcol, 0)
    lax.fori_loop(0, CHUNK, _frow, 0)


def _zero_slice(zbuf, dstref, base):
    full, rem = divmod(ROWS_PT, CHUNK)
    for k in range(full):
        pltpu.sync_copy(zbuf, dstref.at[pl.ds(base + k * CHUNK, CHUNK), :])
    if rem:
        pltpu.sync_copy(zbuf.at[pl.ds(0, rem), :],
                        dstref.at[pl.ds(base + full * CHUNK, rem), :])


def _make_sc_agg(width, with_deg):
    out_type = [jax.ShapeDtypeStruct((NC, N_PAD, 2 * H), jnp.float32)]
    scratch = [
        pltpu.VMEM((QCH, CHUNK), jnp.int32),
        pltpu.VMEM((QCH, CHUNK), jnp.int32),
        pltpu.VMEM((2 * NBURST, CHUNK, width), jnp.float32),
        pltpu.VMEM((1, CHUNK), jnp.int32),
        pltpu.VMEM_SHARED((N_PAD, width), jnp.float32),
        pltpu.VMEM_SHARED((N_PAD, width), jnp.float32),
        pltpu.SemaphoreType.DMA,
        pltpu.SemaphoreType.DMA,
        pltpu.SemaphoreType.DMA,
        pltpu.SemaphoreType.DMA,
    ]
    if with_deg:
        scratch += [pltpu.VMEM((CHUNK, DW), jnp.float32),
                    pltpu.VMEM_SHARED((N_PAD, DW), jnp.float32)]

    @functools.partial(
        pl.kernel,
        mesh=_MESH,
        compiler_params=pltpu.CompilerParams(use_tc_tiling_on_sc=False),
        out_type=out_type,
        scratch_types=scratch,
    )
    def sc_agg(table_hbm, src_hbm, dst_hbm, *args):
        if with_deg:
            (out_hbm, src_q, dst_q, rows, dumidx, acc, tbl,
             gsem0, gsem1, ssem0, ssem1, obuf, dacc) = args
        else:
            (out_hbm, src_q, dst_q, rows, dumidx, acc, tbl,
             gsem0, gsem1, ssem0, ssem1) = args
            obuf = dacc = None
        gsem = (gsem0, gsem1)
        ssem = (ssem0, ssem1)
        c = lax.axis_index("c")
        s = lax.axis_index("s")
        wid = s * NC + c
        base = s * ROWS_PT

        td = pltpu.async_copy(
            table_hbm.at[pl.ds(base, ROWS_PT), pl.ds(0, width)],
            tbl.at[pl.ds(base, ROWS_PT), :], ssem0)

        zbuf = rows.at[0]
        _fill(zbuf, width, 0.0)
        _zero_slice(zbuf, acc, base)
        if with_deg:
            _fill(obuf, DW, 0.0)
            _zero_slice(obuf, dacc, base)
            _fill(obuf, DW, 1.0)

        def _dfill(i, _):
            dumidx[0, pl.ds(i * 16, 16)] = jnp.full((16,), DUMMY, jnp.int32)
            return 0
        lax.fori_loop(0, CHUNK // 16, _dfill, 0)
        td.wait()
        plsc.subcore_barrier()

        def _fire_g(sb, jlo):
            for b in range(NBURST):
                pltpu.async_copy(tbl.at[src_q.at[jlo + b]],
                                 rows.at[NBURST * sb + b], gsem[sb])

        def _drain_g(sb):
            for b in range(NBURST):
                pltpu.make_async_copy(tbl.at[src_q.at[0]],
                                      rows.at[NBURST * sb + b],
                                      gsem[sb]).wait()

        def _fire_s(sb, jlo):
            for b in range(NBURST):
                pltpu.async_copy(rows.at[NBURST * sb + b],
                                 acc.at[dst_q.at[jlo + b]], ssem[sb],
                                 add=True)
            if with_deg:
                for b in range(NBURST):
                    pltpu.async_copy(obuf, dacc.at[dst_q.at[jlo + b]],
                                     ssem[sb], add=True)

        def _fire_dummy_s(sb):
            for b in range(NBURST):
                pltpu.async_copy(rows.at[NBURST * sb + b],
                                 acc.at[dumidx.at[0]], ssem[sb], add=True)
            if with_deg:
                for b in range(NBURST):
                    pltpu.async_copy(obuf, dacc.at[dumidx.at[0]],
                                     ssem[sb], add=True)

        def _drain_s(sb):
            for b in range(NBURST):
                pltpu.make_async_copy(rows.at[NBURST * sb + b],
                                      acc.at[dumidx.at[0]], ssem[sb]).wait()
            if with_deg:
                for b in range(NBURST):
                    pltpu.make_async_copy(obuf, dacc.at[dumidx.at[0]],
                                          ssem[sb]).wait()

        for q in range(CPW // QCH):
            pltpu.sync_copy(
                src_hbm.at[pl.ds(wid * CPW + q * QCH, QCH), :], src_q)
            pltpu.sync_copy(
                dst_hbm.at[pl.ds(wid * CPW + q * QCH, QCH), :], dst_q)
            _fire_g(0, 0)
            _fire_dummy_s(1)

            def _pipe(gg, _):
                c0 = 2 * NBURST * gg
                c2 = c0 + NBURST
                _drain_g(0)
                _fire_s(0, c0)
                _drain_s(1)
                _fire_g(1, c2)
                _drain_g(1)
                _fire_s(1, c2)
                _drain_s(0)
                cn = jnp.minimum(c0 + 2 * NBURST, QCH - NBURST)
                _fire_g(0, cn)
                return 0
            lax.fori_loop(0, QCH // (2 * NBURST), _pipe, 0)
            _drain_g(0)
            _drain_s(1)
        plsc.subcore_barrier()

        pltpu.sync_copy(acc.at[pl.ds(base, ROWS_PT), :],
                        out_hbm.at[c, pl.ds(base, ROWS_PT), pl.ds(0, width)])
        if with_deg:
            pltpu.sync_copy(
                dacc.at[pl.ds(base, ROWS_PT), :],
                out_hbm.at[c, pl.ds(base, ROWS_PT), pl.ds(H, DW)])

    return sc_agg


_sc_agg_l1 = _make_sc_agg(H, True)
_sc_agg_l2 = _make_sc_agg(W2AGG, False)


def kernel(x, edge_index, W1_l, W1_r, b1, W2_l, W2_r, b2):
    pk1, srcp, dstp = _tc1(x, edge_index, W1_l, W1_r, b1.reshape(1, H))
    (so1,) = _sc_agg_l1(pk1, srcp, dstp)
    p2, aux = _tc2(so1, pk1, W2_l, W2_r, b2.reshape(1, 1))
    (s2p,) = _sc_agg_l2(p2, srcp, dstp)
    return _tc3(s2p, aux)


# speedup vs baseline: 1.3060x; 1.3060x over previous
import functools

import jax
import jax.numpy as jnp
from jax import lax
from jax.experimental import pallas as pl
from jax.experimental.pallas import tpu as pltpu
from jax.experimental.pallas import tpu_sc as plsc

N = 10000
E = 320000
IN = 128
H = 64

NC = 2
NS = 16
NW = NC * NS
CHUNK = 128
CPW = 80
E_PAD = NW * CPW * CHUNK
EROWS = E // CHUNK
EROWS_PAD = E_PAD // CHUNK
ROWS_PT = 632
N_PAD = NS * ROWS_PT
DUMMY = N_PAD - 1
W2AGG = 16
DW = 16
NBUF = 4
QCH = 16


def _tc1_body(x_ref, ei_ref, wl_ref, wr_ref, b1_ref, pk1_ref,
              srcp_ref, dstp_ref):
    x = x_ref[...]
    p1 = jnp.dot(x, wl_ref[...], preferred_element_type=jnp.float32)
    r1b = jnp.dot(x, wr_ref[...], preferred_element_type=jnp.float32) + b1_ref[...]
    pk1 = jnp.concatenate([p1, r1b], axis=1)
    pk1_ref[...] = jnp.concatenate(
        [pk1, jnp.zeros((N_PAD - N, 2 * H), jnp.float32)], axis=0)
    ei = ei_ref[...].reshape(2, EROWS, CHUNK)
    srcp_ref[...] = jnp.concatenate(
        [ei[0], jnp.zeros((EROWS_PAD - EROWS, CHUNK), jnp.int32)], axis=0)
    dstp_ref[...] = jnp.concatenate(
        [ei[1], jnp.full((EROWS_PAD - EROWS, CHUNK), DUMMY, jnp.int32)], axis=0)


def _tc1(x, edge_index, W1_l, W1_r, b1):
    return pl.pallas_call(
        _tc1_body,
        out_shape=[
            jax.ShapeDtypeStruct((N_PAD, 2 * H), jnp.float32),
            jax.ShapeDtypeStruct((EROWS_PAD, CHUNK), jnp.int32),
            jax.ShapeDtypeStruct((EROWS_PAD, CHUNK), jnp.int32),
        ],
    )(x, edge_index, W1_l, W1_r, b1)


def _tc2_body(so1_ref, pk1_ref, w2l_ref, w2r_ref, b2_ref, p2_ref, aux_ref):
    s1 = so1_ref[0, :N, :H] + so1_ref[1, :N, :H]
    deg = jnp.maximum(so1_ref[0, :N, H:H + 1] + so1_ref[1, :N, H:H + 1], 1.0)
    rdeg = 1.0 / deg
    h = jnp.maximum(s1 * rdeg + pk1_ref[:N, H:], 0.0)
    p2 = jnp.dot(h, w2l_ref[...], preferred_element_type=jnp.float32)
    r2b = jnp.dot(h, w2r_ref[...], preferred_element_type=jnp.float32) + b2_ref[...]
    block = jnp.concatenate(
        [p2, jnp.zeros((N, 2 * H - 1), jnp.float32)], axis=1)
    p2_ref[...] = jnp.concatenate(
        [block, jnp.zeros((N_PAD - N, 2 * H), jnp.float32)], axis=0)
    aux_ref[...] = jnp.concatenate(
        [rdeg, r2b, jnp.zeros((N, 6), jnp.float32)], axis=1)


def _tc2(so1, pk1, W2_l, W2_r, b2):
    return pl.pallas_call(
        _tc2_body,
        out_shape=[
            jax.ShapeDtypeStruct((N_PAD, 2 * H), jnp.float32),
            jax.ShapeDtypeStruct((N, 8), jnp.float32),
        ],
    )(so1, pk1, W2_l, W2_r, b2)


def _tc3_body(s2p_ref, aux_ref, out_ref):
    s2 = s2p_ref[0, :N, 0:1] + s2p_ref[1, :N, 0:1]
    out_ref[...] = s2 * aux_ref[:, 0:1] + aux_ref[:, 1:2]


def _tc3(s2p, aux):
    return pl.pallas_call(
        _tc3_body,
        out_shape=jax.ShapeDtypeStruct((N, 1), jnp.float32),
    )(s2p, aux)


_MESH = plsc.VectorSubcoreMesh(
    core_axis_name="c", subcore_axis_name="s", num_cores=NC, num_subcores=NS)


def _fill(buf, w, val):
    def _frow(i, _):
        def _fcol(j, _):
            buf[i, pl.ds(j * 16, 16)] = jnp.full((16,), val, jnp.float32)
            return 0
        return lax.fori_loop(0, w // 16, _fcol, 0)
    lax.fori_loop(0, CHUNK, _frow, 0)


def _zero_slice(zbuf, dstref, base):
    full, rem = divmod(ROWS_PT, CHUNK)
    for k in range(full):
        pltpu.sync_copy(zbuf, dstref.at[pl.ds(base + k * CHUNK, CHUNK), :])
    if rem:
        pltpu.sync_copy(zbuf.at[pl.ds(0, rem), :],
                        dstref.at[pl.ds(base + full * CHUNK, rem), :])


def _make_sc_agg(width, with_deg):
    out_type = [jax.ShapeDtypeStruct((NC, N_PAD, 2 * H), jnp.float32)]
    scratch = [
        pltpu.VMEM((QCH, CHUNK), jnp.int32),
        pltpu.VMEM((QCH, CHUNK), jnp.int32),
        pltpu.VMEM((NBUF, CHUNK, width), jnp.float32),
        pltpu.VMEM_SHARED((N_PAD, width), jnp.float32),
        pltpu.VMEM_SHARED((N_PAD, width), jnp.float32),
        pltpu.SemaphoreType.DMA,
        pltpu.SemaphoreType.DMA,
    ]
    if with_deg:
        scratch += [pltpu.VMEM((CHUNK, DW), jnp.float32),
                    pltpu.VMEM_SHARED((N_PAD, DW), jnp.float32)]

    @functools.partial(
        pl.kernel,
        mesh=_MESH,
        compiler_params=pltpu.CompilerParams(use_tc_tiling_on_sc=False),
        out_type=out_type,
        scratch_types=scratch,
    )
    def sc_agg(table_hbm, src_hbm, dst_hbm, *args):
        if with_deg:
            (out_hbm, src_q, dst_q, rows, acc, tbl, gsem, ssem,
             obuf, dacc) = args
        else:
            out_hbm, src_q, dst_q, rows, acc, tbl, gsem, ssem = args
            obuf = dacc = None
        c = lax.axis_index("c")
        s = lax.axis_index("s")
        wid = s * NC + c
        base = s * ROWS_PT

        td = pltpu.async_copy(
            table_hbm.at[pl.ds(base, ROWS_PT), pl.ds(0, width)],
            tbl.at[pl.ds(base, ROWS_PT), :], ssem)

        zbuf = rows.at[0]
        _fill(zbuf, width, 0.0)
        _zero_slice(zbuf, acc, base)
        if with_deg:
            _fill(obuf, DW, 0.0)
            _zero_slice(obuf, dacc, base)
            _fill(obuf, DW, 1.0)

        td.wait()
        plsc.subcore_barrier()

        def _round(q, _):
            qb = wid * CPW + q * QCH
            pltpu.sync_copy(src_hbm.at[pl.ds(qb, QCH), :], src_q)
            pltpu.sync_copy(dst_hbm.at[pl.ds(qb, QCH), :], dst_q)
            gd = [None] * QCH
            sd = [None] * QCH
            for j in range(NBUF):
                gd[j] = pltpu.async_copy(
                    tbl.at[src_q.at[j]], rows.at[j], gsem)
            for j in range(QCH):
                gd[j].wait()
                s = [pltpu.async_copy(
                    rows.at[j % NBUF], acc.at[dst_q.at[j]], ssem, add=True)]
                if with_deg:
                    s.append(pltpu.async_copy(
                        obuf, dacc.at[dst_q.at[j]], ssem, add=True))
                sd[j] = s
                jn = j + NBUF
                if jn < QCH:
                    for d in sd[j]:
                        d.wait()
                    gd[jn] = pltpu.async_copy(
                        tbl.at[src_q.at[jn]], rows.at[jn % NBUF], gsem)
            for j in range(QCH - NBUF, QCH):
                for d in sd[j]:
                    d.wait()
            return 0
        lax.fori_loop(0, CPW // QCH, _round, 0)
        plsc.subcore_barrier()

        pltpu.sync_copy(acc.at[pl.ds(base, ROWS_PT), :],
                        out_hbm.at[c, pl.ds(base, ROWS_PT), pl.ds(0, width)])
        if with_deg:
            pltpu.sync_copy(
                dacc.at[pl.ds(base, ROWS_PT), :],
                out_hbm.at[c, pl.ds(base, ROWS_PT), pl.ds(H, DW)])

    return sc_agg


_sc_agg_l1 = _make_sc_agg(H, True)
_sc_agg_l2 = _make_sc_agg(W2AGG, False)


def kernel(x, edge_index, W1_l, W1_r, b1, W2_l, W2_r, b2):
    pk1, srcp, dstp = _tc1(x, edge_index, W1_l, W1_r, b1.reshape(1, H))
    (so1,) = _sc_agg_l1(pk1, srcp, dstp)
    p2, aux = _tc2(so1, pk1, W2_l, W2_r, b2.reshape(1, 1))
    (s2p,) = _sc_agg_l2(p2, srcp, dstp)
    return _tc3(s2p, aux)


# speedup vs baseline: 1.3277x; 1.0167x over previous
import functools

import jax
import jax.numpy as jnp
from jax import lax
from jax.experimental import pallas as pl
from jax.experimental.pallas import tpu as pltpu
from jax.experimental.pallas import tpu_sc as plsc

N = 10000
E = 320000
IN = 128
H = 64

NC = 2
NS = 16
NW = NC * NS
CHUNK = 128
CPW = 80
E_PAD = NW * CPW * CHUNK
EROWS = E // CHUNK
EROWS_PAD = E_PAD // CHUNK
ROWS_PT = 632
N_PAD = NS * ROWS_PT
DUMMY = N_PAD - 1
W2AGG = 16
DW = 16
NBUF = 4
QCH = 16


def _tc1_body(x_ref, ei_ref, wl_ref, wr_ref, b1_ref, pk1_ref,
              srcp_ref, dstp_ref):
    x = x_ref[...]
    p1 = jnp.dot(x, wl_ref[...], preferred_element_type=jnp.float32)
    r1b = (jnp.dot(x, wr_ref[...], preferred_element_type=jnp.float32)
           + b1_ref[...][None, :])
    pk1 = jnp.concatenate([p1, r1b], axis=1)
    pk1_ref[...] = jnp.concatenate(
        [pk1, jnp.zeros((N_PAD - N, 2 * H), jnp.float32)], axis=0)
    ei = ei_ref[...].reshape(2, EROWS, CHUNK)
    srcp_ref[...] = jnp.concatenate(
        [ei[0], jnp.zeros((EROWS_PAD - EROWS, CHUNK), jnp.int32)], axis=0)
    dstp_ref[...] = jnp.concatenate(
        [ei[1], jnp.full((EROWS_PAD - EROWS, CHUNK), DUMMY, jnp.int32)], axis=0)


def _tc1(x, edge_index, W1_l, W1_r, b1):
    return pl.pallas_call(
        _tc1_body,
        out_shape=[
            jax.ShapeDtypeStruct((N_PAD, 2 * H), jnp.float32),
            jax.ShapeDtypeStruct((EROWS_PAD, CHUNK), jnp.int32),
            jax.ShapeDtypeStruct((EROWS_PAD, CHUNK), jnp.int32),
        ],
    )(x, edge_index, W1_l, W1_r, b1)


def _tc2_body(so1_ref, pk1_ref, w2l_ref, w2r_ref, b2_ref, p2_ref, aux_ref):
    s1 = so1_ref[0, :N, :H] + so1_ref[1, :N, :H]
    deg = jnp.maximum(so1_ref[0, :N, H:H + 1] + so1_ref[1, :N, H:H + 1], 1.0)
    rdeg = 1.0 / deg
    h = jnp.maximum(s1 * rdeg + pk1_ref[:N, H:], 0.0)
    p2 = jnp.dot(h, w2l_ref[...], preferred_element_type=jnp.float32)
    r2b = (jnp.dot(h, w2r_ref[...], preferred_element_type=jnp.float32)
           + b2_ref[...][None, :])
    block = jnp.concatenate(
        [p2, jnp.zeros((N, 2 * H - 1), jnp.float32)], axis=1)
    p2_ref[...] = jnp.concatenate(
        [block, jnp.zeros((N_PAD - N, 2 * H), jnp.float32)], axis=0)
    aux_ref[...] = jnp.concatenate(
        [rdeg, r2b, jnp.zeros((N, 6), jnp.float32)], axis=1)


def _tc2(so1, pk1, W2_l, W2_r, b2):
    return pl.pallas_call(
        _tc2_body,
        out_shape=[
            jax.ShapeDtypeStruct((N_PAD, 2 * H), jnp.float32),
            jax.ShapeDtypeStruct((N, 8), jnp.float32),
        ],
    )(so1, pk1, W2_l, W2_r, b2)


def _tc3_body(s2p_ref, aux_ref, out_ref):
    s2 = s2p_ref[0, :N, 0:1] + s2p_ref[1, :N, 0:1]
    out_ref[...] = s2 * aux_ref[:, 0:1] + aux_ref[:, 1:2]


def _tc3(s2p, aux):
    return pl.pallas_call(
        _tc3_body,
        out_shape=jax.ShapeDtypeStruct((N, 1), jnp.float32),
    )(s2p, aux)


_MESH = plsc.VectorSubcoreMesh(
    core_axis_name="c", subcore_axis_name="s", num_cores=NC, num_subcores=NS)


def _fill(buf, w, val):
    def _frow(i, _):
        def _fcol(j, _):
            buf[i, pl.ds(j * 16, 16)] = jnp.full((16,), val, jnp.float32)
            return 0
        return lax.fori_loop(0, w // 16, _fcol, 0)
    lax.fori_loop(0, CHUNK, _frow, 0)


def _zero_slice(zbuf, dstref, base):
    full, rem = divmod(ROWS_PT, CHUNK)
    for k in range(full):
        pltpu.sync_copy(zbuf, dstref.at[pl.ds(base + k * CHUNK, CHUNK), :])
    if rem:
        pltpu.sync_copy(zbuf.at[pl.ds(0, rem), :],
                        dstref.at[pl.ds(base + full * CHUNK, rem), :])


def _make_sc_agg(width, with_deg, nbuf, qch):
    out_type = [jax.ShapeDtypeStruct((NC, N_PAD, 2 * H), jnp.float32)]
    scratch = [
        pltpu.VMEM((qch, CHUNK), jnp.int32),
        pltpu.VMEM((qch, CHUNK), jnp.int32),
        pltpu.VMEM((nbuf, CHUNK, width), jnp.float32),
        pltpu.VMEM_SHARED((N_PAD, width), jnp.float32),
        pltpu.VMEM_SHARED((N_PAD, width), jnp.float32),
        pltpu.SemaphoreType.DMA,
        pltpu.SemaphoreType.DMA,
    ]
    if with_deg:
        scratch += [pltpu.VMEM((CHUNK, DW), jnp.float32),
                    pltpu.VMEM_SHARED((N_PAD, DW), jnp.float32)]

    @functools.partial(
        pl.kernel,
        mesh=_MESH,
        compiler_params=pltpu.CompilerParams(use_tc_tiling_on_sc=False),
        out_type=out_type,
        scratch_types=scratch,
    )
    def sc_agg(table_hbm, src_hbm, dst_hbm, *args):
        if with_deg:
            (out_hbm, src_q, dst_q, rows, acc, tbl, gsem, ssem,
             obuf, dacc) = args
        else:
            out_hbm, src_q, dst_q, rows, acc, tbl, gsem, ssem = args
            obuf = dacc = None
        c = lax.axis_index("c")
        s = lax.axis_index("s")
        wid = s * NC + c
        base = s * ROWS_PT

        td = pltpu.async_copy(
            table_hbm.at[pl.ds(base, ROWS_PT), pl.ds(0, width)],
            tbl.at[pl.ds(base, ROWS_PT), :], ssem)

        zbuf = rows.at[0]
        _fill(zbuf, width, 0.0)
        _zero_slice(zbuf, acc, base)
        if with_deg:
            _fill(obuf, DW, 0.0)
            _zero_slice(obuf, dacc, base)
            _fill(obuf, DW, 1.0)

        td.wait()
        plsc.subcore_barrier()

        def _round(q, _):
            qb = wid * CPW + q * qch
            pltpu.sync_copy(src_hbm.at[pl.ds(qb, qch), :], src_q)
            pltpu.sync_copy(dst_hbm.at[pl.ds(qb, qch), :], dst_q)
            gd = [None] * qch
            sd = [None] * qch
            for j in range(nbuf):
                gd[j] = pltpu.async_copy(
                    tbl.at[src_q.at[j]], rows.at[j], gsem)
            for j in range(qch):
                gd[j].wait()
                s = [pltpu.async_copy(
                    rows.at[j % nbuf], acc.at[dst_q.at[j]], ssem, add=True)]
                if with_deg:
                    s.append(pltpu.async_copy(
                        obuf, dacc.at[dst_q.at[j]], ssem, add=True))
                sd[j] = s
                jn = j + nbuf
                if jn < qch:
                    for d in sd[j]:
                        d.wait()
                    gd[jn] = pltpu.async_copy(
                        tbl.at[src_q.at[jn]], rows.at[jn % nbuf], gsem)
            for j in range(qch - nbuf, qch):
                for d in sd[j]:
                    d.wait()
            return 0
        lax.fori_loop(0, CPW // qch, _round, 0)
        plsc.subcore_barrier()

        pltpu.sync_copy(acc.at[pl.ds(base, ROWS_PT), :],
                        out_hbm.at[c, pl.ds(base, ROWS_PT), pl.ds(0, width)])
        if with_deg:
            pltpu.sync_copy(
                dacc.at[pl.ds(base, ROWS_PT), :],
                out_hbm.at[c, pl.ds(base, ROWS_PT), pl.ds(H, DW)])

    return sc_agg


_sc_agg_l1 = _make_sc_agg(H, True, NBUF, QCH)
_sc_agg_l2 = _make_sc_agg(W2AGG, False, 8, 40)


def kernel(x, edge_index, W1_l, W1_r, b1, W2_l, W2_r, b2):
    pk1, srcp, dstp = _tc1(x, edge_index, W1_l, W1_r, b1)
    (so1,) = _sc_agg_l1(pk1, srcp, dstp)
    p2, aux = _tc2(so1, pk1, W2_l, W2_r, b2)
    (s2p,) = _sc_agg_l2(p2, srcp, dstp)
    return _tc3(s2p, aux)
